# Initial kernel scaffold; baseline (speedup 1.0000x reference)
import jax
import jax.numpy as jnp
from jax.experimental import pallas as pl


def kernel(embeddings, pos_table):
    raise NotImplementedError("write your pallas kernel here")


# speedup vs baseline: 1.6689x; 1.6689x over previous
import jax
import jax.numpy as jnp
from jax.experimental import pallas as pl


_TS = 512


def _body(emb_ref, pos_ref, out_ref):
    out_ref[0] = emb_ref[0] + pos_ref[...]


def kernel(embeddings, pos_table):
    B, S, E = embeddings.shape
    grid = (S // _TS, B)
    return pl.pallas_call(
        _body,
        grid=grid,
        in_specs=[
            pl.BlockSpec((1, _TS, E), lambda s, b: (b, s, 0)),
            pl.BlockSpec((_TS, E), lambda s, b: (s, 0)),
        ],
        out_specs=pl.BlockSpec((1, _TS, E), lambda s, b: (b, s, 0)),
        out_shape=jax.ShapeDtypeStruct((B, S, E), embeddings.dtype),
    )(embeddings, pos_table)
